# baseline (speedup 1.0000x reference)
import functools

import jax
import jax.numpy as jnp
from jax import lax
from jax.experimental import pallas as pl
from jax.experimental.pallas import tpu as pltpu
from jax.experimental.pallas import tpu_sc as plsc

B, R, N = 128, 4, 65536
B_BLK = 8
ALLOC_C = 1.0 / N

_info = plsc.get_sparse_core_info()
_NC, _NS = _info.num_cores, _info.num_subcores
_NW = _NC * _NS
_TOTAL = B * N
_PER_W = _TOTAL // _NW
_CHUNK = 32768
_N_DMA = _PER_W // _CHUNK


def _phi_kernel(fg_ref, rw_ref, phi_ref):
    for b in range(B_BLK):
        p = 1.0 - fg_ref[b, 0] * rw_ref[b, 0, :]
        p = p * (1.0 - fg_ref[b, 1] * rw_ref[b, 1, :])
        p = p * (1.0 - fg_ref[b, 2] * rw_ref[b, 2, :])
        p = p * (1.0 - fg_ref[b, 3] * rw_ref[b, 3, :])
        phi_ref[b, :] = p


@functools.partial(
    pl.kernel,
    mesh=plsc.VectorSubcoreMesh(core_axis_name="c", subcore_axis_name="s"),
    out_type=jax.ShapeDtypeStruct((_TOTAL,), jnp.float32),
    scratch_types=[pltpu.VMEM((_CHUNK,), jnp.float32)],
)
def _alloc_fill(out_hbm, buf):
    wid = lax.axis_index("s") * _NC + lax.axis_index("c")

    def body(i, carry):
        buf[pl.ds(i * 16, 16)] = jnp.full((16,), ALLOC_C, dtype=jnp.float32)
        return carry

    lax.fori_loop(0, _CHUNK // 16, body, 0)
    base = wid * _PER_W
    for j in range(_N_DMA):
        pltpu.sync_copy(buf, out_hbm.at[pl.ds(base + j * _CHUNK, _CHUNK)])


def kernel(write_weights, read_weights, free_gates, write_gate, diff_alloc):
    del write_weights, write_gate, diff_alloc
    grid = (B // B_BLK,)
    alloc = _alloc_fill().reshape(B, N)
    phi = pl.pallas_call(
        _phi_kernel,
        grid=grid,
        in_specs=[
            pl.BlockSpec((B_BLK, R), lambda i: (i, 0), memory_space=pltpu.SMEM),
            pl.BlockSpec((B_BLK, R, N), lambda i: (i, 0, 0)),
        ],
        out_specs=pl.BlockSpec((B_BLK, N), lambda i: (i, 0)),
        out_shape=jax.ShapeDtypeStruct((B, N), jnp.float32),
    )(free_gates, read_weights)
    return (alloc, phi)


# speedup vs baseline: 1.6048x; 1.6048x over previous
import jax
import jax.numpy as jnp
from jax.experimental import pallas as pl
from jax.experimental.pallas import tpu as pltpu

B, R, N = 128, 4, 65536
B_BLK = 8
N_SPLIT = 2
N_BLK = N // N_SPLIT


def _phi_kernel(fg_ref, rw_ref, phi_ref, alloc_ref):
    for b in range(B_BLK):
        p = 1.0 - fg_ref[b, 0] * rw_ref[b, 0, :]
        p = p * (1.0 - fg_ref[b, 1] * rw_ref[b, 1, :])
        p = p * (1.0 - fg_ref[b, 2] * rw_ref[b, 2, :])
        p = p * (1.0 - fg_ref[b, 3] * rw_ref[b, 3, :])
        phi_ref[b, :] = p
    alloc_ref[...] = jnp.full(alloc_ref.shape, 1.0 / N, dtype=jnp.float32)


def kernel(write_weights, read_weights, free_gates, write_gate, diff_alloc):
    del write_weights, write_gate, diff_alloc
    grid = (B // B_BLK, N_SPLIT)
    phi, alloc = pl.pallas_call(
        _phi_kernel,
        grid=grid,
        in_specs=[
            pl.BlockSpec((B_BLK, R), lambda i, j: (i, 0), memory_space=pltpu.SMEM),
            pl.BlockSpec((B_BLK, R, N_BLK), lambda i, j: (i, 0, j)),
        ],
        out_specs=[
            pl.BlockSpec((B_BLK, N_BLK), lambda i, j: (i, j)),
            pl.BlockSpec((B_BLK, N_BLK), lambda i, j: (i, j)),
        ],
        out_shape=[
            jax.ShapeDtypeStruct((B, N), jnp.float32),
            jax.ShapeDtypeStruct((B, N), jnp.float32),
        ],
    )(free_gates, read_weights)
    return (alloc, phi)
